# baseline (speedup 1.0000x reference)
import functools

import jax
import jax.numpy as jnp
from jax import lax
from jax.experimental import pallas as pl
from jax.experimental.pallas import tpu as pltpu
from jax.experimental.pallas import tpu_sc as plsc

N = 10000
E = 160000
D_IN = 256
D_H = 512
N_CLS = 40

W = 256
SZH = E + W
R = 2 * SZH
CW = 128
NCHUNK = D_H // CW
NC2 = NCHUNK // 2
NH = 5120
HALF = 5000
ND = NH - HALF
BN = 1000
HIST = 10240
ETI = E // 16
ER = 1250

_mesh = plsc.VectorSubcoreMesh(core_axis_name="c", subcore_axis_name="s")


def _pos_kernel(dst_ref, pos_ref, loc_ref, nw_ref):
    d = dst_ref[...]
    m0 = d < HALF
    a = m0.astype(jnp.float32)
    col = lax.broadcasted_iota(jnp.int32, (ER, 128), 1)
    row = lax.broadcasted_iota(jnp.int32, (ER, 128), 0)
    i_ = lax.broadcasted_iota(jnp.int32, (128, 128), 0)
    j_ = lax.broadcasted_iota(jnp.int32, (128, 128), 1)
    lt = (i_ < j_).astype(jnp.float32)
    in_excl = jnp.dot(a, lt, preferred_element_type=jnp.float32)
    rs = jnp.sum(a, axis=1, keepdims=True)
    ri = lax.broadcasted_iota(jnp.int32, (ER, ER), 0)
    rj = lax.broadcasted_iota(jnp.int32, (ER, ER), 1)
    rlt = (ri > rj).astype(jnp.float32)
    row_excl = jnp.dot(rlt, rs, preferred_element_type=jnp.float32)
    p0 = (in_excl + row_excl).astype(jnp.int32)
    eidx = row * 128 + col
    pos_ref[...] = jnp.where(m0, p0, SZH + eidx - p0)
    loc_ref[...] = jnp.where(m0, d, d - HALF)
    cnt0 = jnp.sum(a).astype(jnp.int32)
    nw0 = ((cnt0 + (W - 1)) >> 8).astype(jnp.float32)
    nw1 = (((E - cnt0) + (W - 1)) >> 8).astype(jnp.float32)
    li = lax.broadcasted_iota(jnp.int32, (1, 128), 1)
    nw_ref[...] = jnp.where(li == 0, nw0, jnp.where(li == 1, nw1, 0.0))


def _pos_prep(dst2d):
    return pl.pallas_call(
        _pos_kernel,
        grid=(1,),
        in_specs=[pl.BlockSpec((ER, 128), lambda i: (0, 0))],
        out_specs=[
            pl.BlockSpec((ER, 128), lambda i: (0, 0)),
            pl.BlockSpec((ER, 128), lambda i: (0, 0)),
            pl.BlockSpec((1, 128), lambda i: (0, 0)),
        ],
        out_shape=[
            jax.ShapeDtypeStruct((ER, 128), jnp.int32),
            jax.ShapeDtypeStruct((ER, 128), jnp.int32),
            jax.ShapeDtypeStruct((1, 128), jnp.float32),
        ],
    )(dst2d)


@functools.partial(
    pl.kernel,
    out_type=[
        jax.ShapeDtypeStruct((R,), jnp.int32),
        jax.ShapeDtypeStruct((R,), jnp.int32),
        jax.ShapeDtypeStruct((HIST,), jnp.float32),
    ],
    mesh=_mesh,
    scratch_types=[
        pltpu.VMEM((ETI,), jnp.int32),
        pltpu.VMEM((ETI,), jnp.int32),
        pltpu.VMEM((ETI,), jnp.int32),
        pltpu.VMEM((ETI,), jnp.int32),
        pltpu.VMEM((ETI,), jnp.float32),
        pltpu.VMEM((HIST // 16,), jnp.float32),
        pltpu.VMEM((2048,), jnp.int32),
        pltpu.VMEM((2048,), jnp.int32),
        pltpu.VMEM_SHARED((HIST,), jnp.float32),
        pltpu.SemaphoreType.DMA,
        pltpu.SemaphoreType.DMA,
    ],
)
def _part_kernel(src_hbm, dst_hbm, pos_hbm, loc_hbm, psrc_hbm, pdst_hbm,
                 deg_hbm, src_v, dst_v, pos_v, loc_v, ones_v, zh_v,
                 z2_v, dp_v, hacc, sem, dsem):
    c = lax.axis_index("c")
    s = lax.axis_index("s")
    lane = lax.iota(jnp.int32, 16)

    @pl.when(c == 0)
    def _():
        pltpu.sync_copy(src_hbm.at[pl.ds(s * ETI, ETI)], src_v)
        pltpu.sync_copy(dst_hbm.at[pl.ds(s * ETI, ETI)], dst_v)
        pltpu.sync_copy(pos_hbm.at[pl.ds(s * ETI, ETI)], pos_v)
        pltpu.sync_copy(loc_hbm.at[pl.ds(s * ETI, ETI)], loc_v)

        def fill_ones(k, _):
            ones_v[pl.ds(k * 16, 16)] = jnp.ones((16,), jnp.float32)
            return 0

        lax.fori_loop(0, ETI // 16, fill_ones, 0)

        def fill_zero(k, _):
            zh_v[pl.ds(k * 16, 16)] = jnp.zeros((16,), jnp.float32)
            return 0

        lax.fori_loop(0, HIST // 256, fill_zero, 0)

        def fill_pat(k, _):
            z2_v[pl.ds(k * 16, 16)] = jnp.zeros((16,), jnp.int32)
            dp_v[pl.ds(k * 16, 16)] = HALF + ((k * 16 + lane) % ND)
            return 0

        lax.fori_loop(0, 128, fill_pat, 0)

        pltpu.sync_copy(zh_v, hacc.at[pl.ds(s * (HIST // 16), HIST // 16)])

        for k in range(9):
            pltpu.sync_copy(z2_v,
                            psrc_hbm.at[pl.ds(s * (R // 16) + k * 2048,
                                              2048)])
            pltpu.sync_copy(dp_v,
                            pdst_hbm.at[pl.ds(s * (R // 16) + k * 2048,
                                              2048)])
        pltpu.sync_copy(z2_v.at[pl.ds(0, R // 16 - 9 * 2048)],
                        psrc_hbm.at[pl.ds(s * (R // 16) + 9 * 2048,
                                          R // 16 - 9 * 2048)])
        pltpu.sync_copy(dp_v.at[pl.ds(0, R // 16 - 9 * 2048)],
                        pdst_hbm.at[pl.ds(s * (R // 16) + 9 * 2048,
                                          R // 16 - 9 * 2048)])
        plsc.subcore_barrier()

        hdma = pltpu.async_copy(ones_v, hacc.at[dst_v], dsem, add=True)
        pltpu.async_copy(src_v, psrc_hbm.at[pos_v], sem).wait()
        pltpu.async_copy(loc_v, pdst_hbm.at[pos_v], sem).wait()
        hdma.wait()
        plsc.subcore_barrier()

        pltpu.sync_copy(hacc.at[pl.ds(s * (HIST // 16), HIST // 16)],
                        deg_hbm.at[pl.ds(s * (HIST // 16), HIST // 16)])


@functools.partial(
    pl.kernel,
    out_type=jax.ShapeDtypeStruct((NCHUNK * N, CW), jnp.float32),
    mesh=_mesh,
    scratch_types=[
        pltpu.VMEM((W,), jnp.int32),
        pltpu.VMEM((W,), jnp.int32),
        pltpu.VMEM((W,), jnp.int32),
        pltpu.VMEM((W,), jnp.int32),
        pltpu.VMEM((W, CW), jnp.float32),
        pltpu.VMEM((W, CW), jnp.float32),
        pltpu.VMEM((128,), jnp.float32),
        pltpu.VMEM_SHARED((NH, CW), jnp.float32),
        pltpu.SemaphoreType.DMA,
        pltpu.SemaphoreType.DMA,
        pltpu.SemaphoreType.DMA,
    ],
)
def _prop_kernel(z_hbm, psrc4_hbm, pdst_hbm, nw_hbm, zrows_hbm, y_hbm,
                 src_v0, src_v1, dst_v0, dst_v1, rows_v0, rows_v1, nw_v,
                 acc, isem, gsem, ssem):
    c = lax.axis_index("c")
    s = lax.axis_index("s")
    srcb = (src_v0, src_v1)
    dstb = (dst_v0, dst_v1)
    rowsb = (rows_v0, rows_v1)

    pltpu.sync_copy(nw_hbm, nw_v)
    nwvec = nw_v[pl.ds(0, 16)]
    nw_h = (nwvec[0].astype(jnp.int32), nwvec[1].astype(jnp.int32))

    def idx_issue(chunk, hbase, w, b):
        eoff = hbase + w * W
        pltpu.async_copy(psrc4_hbm.at[pl.ds(chunk * R + eoff, W)],
                         srcb[b], isem)
        pltpu.async_copy(pdst_hbm.at[pl.ds(eoff, W)], dstb[b], isem)

    def idx_wait():
        pltpu.make_async_copy(psrc4_hbm.at[pl.ds(0, W)], srcb[0],
                              isem).wait()
        pltpu.make_async_copy(pdst_hbm.at[pl.ds(0, W)], dstb[0], isem).wait()

    def scat_wait():
        pltpu.make_async_copy(rowsb[0], acc.at[pl.ds(0, W)], ssem).wait()

    first = True
    for j in range(NC2):
        chunk = c * NC2 + j
        for h in range(2):
            hbase = h * SZH
            nw = nw_h[h]
            if not first:
                plsc.subcore_barrier()
            first = False

            @pl.when(s < 15)
            def _():
                pltpu.sync_copy(
                    z_hbm.at[pl.ds(chunk * N + h * HALF + s * 312, 312)],
                    acc.at[pl.ds(s * 312, 312)])

            @pl.when(s == 15)
            def _():
                pltpu.sync_copy(
                    z_hbm.at[pl.ds(chunk * N + h * HALF + 4680, 320)],
                    acc.at[pl.ds(4680, 320)])
                pltpu.sync_copy(zrows_hbm, acc.at[pl.ds(HALF, ND)])

            plsc.subcore_barrier()

            nb = jnp.maximum(0, (nw - s + 15) >> 4)

            @pl.when(nb >= 1)
            def _():
                idx_issue(chunk, hbase, s, 0)

            def win(ii, i_dyn, b):
                @pl.when(ii < nb)
                def _():
                    idx_wait()
                    pltpu.async_copy(z_hbm.at[srcb[b]], rowsb[b],
                                     gsem).wait()
                    pltpu.async_copy(rowsb[b], acc.at[dstb[b]], ssem,
                                     add=True)

                    @pl.when((ii >= 1) & (ii + 1 < nb))
                    def _():
                        scat_wait()

                    @pl.when(ii + 1 < nb)
                    def _():
                        idx_issue(chunk, hbase, s + 16 * (i_dyn + 1), 1 - b)

            def pair(t, _):
                win(2 * t, 2 * t, 0)
                win(2 * t + 1, 2 * t + 1, 1)
                return 0

            lax.fori_loop(0, ((SZH // W) // 16 + 1 + 1) // 2 + 1, pair, 0)

            @pl.when(nb >= 1)
            def _():
                scat_wait()

            @pl.when(nb >= 2)
            def _():
                scat_wait()

            plsc.subcore_barrier()

            @pl.when(s < 15)
            def _():
                pltpu.sync_copy(
                    acc.at[pl.ds(s * 312, 312)],
                    y_hbm.at[pl.ds(chunk * N + h * HALF + s * 312, 312)])

            @pl.when(s == 15)
            def _():
                pltpu.sync_copy(
                    acc.at[pl.ds(4680, 320)],
                    y_hbm.at[pl.ds(chunk * N + h * HALF + 4680, 320)])


def _tc1_kernel(x_ref, w_ref, b_ref, deg_ref, z_ref, dinv_ref):
    deg = deg_ref[...] + 1.0
    dinv = lax.rsqrt(jnp.maximum(deg, 1e-6))
    acc = jnp.dot(x_ref[...], w_ref[0], preferred_element_type=jnp.float32)
    z_ref[0] = dinv * (acc + b_ref[0])
    dinv_ref[...] = dinv


def _tc1(x, w, b, deg):
    return pl.pallas_call(
        _tc1_kernel,
        grid=(N // BN, NCHUNK),
        in_specs=[
            pl.BlockSpec((BN, D_IN), lambda i, cc: (i, 0)),
            pl.BlockSpec((1, D_IN, CW), lambda i, cc: (cc, 0, 0)),
            pl.BlockSpec((1, 1, CW), lambda i, cc: (cc, 0, 0)),
            pl.BlockSpec((BN, 1), lambda i, cc: (i, 0)),
        ],
        out_specs=[
            pl.BlockSpec((1, BN, CW), lambda i, cc: (cc, i, 0)),
            pl.BlockSpec((BN, 1), lambda i, cc: (i, 0)),
        ],
        out_shape=[
            jax.ShapeDtypeStruct((NCHUNK, N, CW), jnp.float32),
            jax.ShapeDtypeStruct((N, 1), jnp.float32),
        ],
    )(x, w, b, deg)


def _tc2_kernel(y_ref, w_ref, b_ref, dinv_ref, z_ref):
    k = pl.program_id(2)
    dinv = dinv_ref[...]
    h = jnp.maximum(dinv * y_ref[0], 0.0)
    part = jnp.dot(h, w_ref[0, 0], preferred_element_type=jnp.float32)

    @pl.when(k == 0)
    def _():
        z_ref[0] = part + b_ref[0]

    @pl.when(k > 0)
    def _():
        z_ref[0] += part

    @pl.when(k == NCHUNK - 1)
    def _():
        z_ref[0] *= dinv


def _tc2(y, w, b, dinv):
    return pl.pallas_call(
        _tc2_kernel,
        grid=(N // BN, NCHUNK, NCHUNK),
        in_specs=[
            pl.BlockSpec((1, BN, CW), lambda i, co, k: (k, i, 0)),
            pl.BlockSpec((1, 1, CW, CW), lambda i, co, k: (k, co, 0, 0)),
            pl.BlockSpec((1, 1, CW), lambda i, co, k: (co, 0, 0)),
            pl.BlockSpec((BN, 1), lambda i, co, k: (i, 0)),
        ],
        out_specs=pl.BlockSpec((1, BN, CW), lambda i, co, k: (co, i, 0)),
        out_shape=jax.ShapeDtypeStruct((NCHUNK, N, CW), jnp.float32),
    )(y, w, b, dinv)


def _tc3_kernel(y_ref, w_ref, b_ref, dinv_ref, o_ref):
    k = pl.program_id(1)
    h = jnp.maximum(dinv_ref[...] * y_ref[0], 0.0)
    part = jnp.dot(h, w_ref[0], preferred_element_type=jnp.float32)

    @pl.when(k == 0)
    def _():
        o_ref[...] = part + b_ref[...]

    @pl.when(k > 0)
    def _():
        o_ref[...] += part


def _tc3(y, w, b, dinv):
    return pl.pallas_call(
        _tc3_kernel,
        grid=(N // BN, NCHUNK),
        in_specs=[
            pl.BlockSpec((1, BN, CW), lambda i, k: (k, i, 0)),
            pl.BlockSpec((1, CW, N_CLS), lambda i, k: (k, 0, 0)),
            pl.BlockSpec((1, N_CLS), lambda i, k: (0, 0)),
            pl.BlockSpec((BN, 1), lambda i, k: (i, 0)),
        ],
        out_specs=pl.BlockSpec((BN, N_CLS), lambda i, k: (i, 0)),
        out_shape=jax.ShapeDtypeStruct((N, N_CLS), jnp.float32),
    )(y, w, b, dinv)


def kernel(x, edge_index, W1, b1, W2, b2, Wd, bd):
    src = edge_index[0].astype(jnp.int32)
    dst = edge_index[1].astype(jnp.int32)

    pos2d, loc2d, nw2d = _pos_prep(dst.reshape(ER, 128))
    psrc, pdst, deg = _part_kernel(src, dst, pos2d.reshape(E),
                                   loc2d.reshape(E))
    nw = nw2d.reshape(128)
    offs = (jnp.arange(NCHUNK, dtype=jnp.int32) * N)[:, None]
    psrc4 = (psrc[None, :] + offs).reshape(-1)
    zrows = jnp.zeros((ND, CW), jnp.float32)

    W1r = W1.reshape(D_IN, NCHUNK, CW).transpose(1, 0, 2)
    W2r = W2.reshape(NCHUNK, CW, NCHUNK, CW).transpose(0, 2, 1, 3)

    deg = deg[:N].reshape(N, 1)
    z1, dinv = _tc1(x, W1r, b1.reshape(NCHUNK, 1, CW), deg)
    y1 = _prop_kernel(z1.reshape(NCHUNK * N, CW), psrc4, pdst, nw, zrows)
    z2 = _tc2(y1.reshape(NCHUNK, N, CW), W2r, b2.reshape(NCHUNK, 1, CW), dinv)
    y2 = _prop_kernel(z2.reshape(NCHUNK * N, CW), psrc4, pdst, nw, zrows)
    logits = _tc3(y2.reshape(NCHUNK, N, CW), Wd.reshape(NCHUNK, CW, N_CLS),
                  bd.reshape(1, N_CLS), dinv)
    return logits


# speedup vs baseline: 1.3945x; 1.3945x over previous
import functools

import jax
import jax.numpy as jnp
from jax import lax
from jax.experimental import pallas as pl
from jax.experimental.pallas import tpu as pltpu
from jax.experimental.pallas import tpu_sc as plsc

N = 10000
E = 160000
D_IN = 256
D_H = 512
N_CLS = 40

EP = 163840
W = 320
CW = 128
NCHUNK = D_H // CW
NC2 = NCHUNK // 2
NH = 5120
HALF = 5000
BN = 1000
HIST = 10240

_mesh = plsc.VectorSubcoreMesh(core_axis_name="c", subcore_axis_name="s")


@functools.partial(
    pl.kernel,
    out_type=jax.ShapeDtypeStruct((HIST,), jnp.float32),
    mesh=_mesh,
    scratch_types=[
        pltpu.VMEM((E // 16,), jnp.int32),
        pltpu.VMEM((E // 16,), jnp.float32),
        pltpu.VMEM((HIST // 16,), jnp.float32),
        pltpu.VMEM_SHARED((HIST,), jnp.float32),
        pltpu.SemaphoreType.DMA,
    ],
)
def _deg_kernel(dst_hbm, out_hbm, idx_v, ones_v, z_v, dacc, sem):
    c = lax.axis_index("c")
    s = lax.axis_index("s")
    npt = HIST // 16
    ept = E // 16

    def fill_ones(k, _):
        ones_v[pl.ds(k * 16, 16)] = jnp.ones((16,), jnp.float32)
        return 0

    lax.fori_loop(0, ept // 16, fill_ones, 0)

    def fill_zero(k, _):
        z_v[pl.ds(k * 16, 16)] = jnp.zeros((16,), jnp.float32)
        return 0

    lax.fori_loop(0, npt // 16, fill_zero, 0)
    pltpu.sync_copy(z_v, dacc.at[pl.ds(s * npt, npt)])
    plsc.subcore_barrier()
    pltpu.sync_copy(dst_hbm.at[pl.ds(s * ept, ept)], idx_v)
    pltpu.async_copy(ones_v, dacc.at[idx_v], sem, add=True).wait()
    plsc.subcore_barrier()

    @pl.when(c == 0)
    def _():
        pltpu.sync_copy(dacc.at[pl.ds(s * npt, npt)],
                        out_hbm.at[pl.ds(s * npt, npt)])


@functools.partial(
    pl.kernel,
    out_type=jax.ShapeDtypeStruct((NCHUNK * N, CW), jnp.float32),
    mesh=_mesh,
    scratch_types=[
        pltpu.VMEM((W,), jnp.int32),
        pltpu.VMEM((W,), jnp.int32),
        pltpu.VMEM((W,), jnp.int32),
        pltpu.VMEM((W,), jnp.int32),
        pltpu.VMEM((W, CW), jnp.float32),
        pltpu.VMEM((W, CW), jnp.float32),
        pltpu.VMEM_SHARED((NH, CW), jnp.float32),
        pltpu.SemaphoreType.DMA,
        pltpu.SemaphoreType.DMA,
        pltpu.SemaphoreType.DMA,
    ],
)
def _prop_kernel(z_hbm, src4_hbm, dh0_hbm, dh1_hbm, zrows_hbm, y_hbm,
                 src_v0, src_v1, dst_v0, dst_v1, rows_v0, rows_v1, acc,
                 isem, gsem, ssem):
    c = lax.axis_index("c")
    s = lax.axis_index("s")
    ept = EP // 16
    nwin = ept // W

    first = True
    for j in range(NC2):
        chunk = c * NC2 + j
        for h in range(2):
            dh_hbm = dh0_hbm if h == 0 else dh1_hbm
            if not first:
                plsc.subcore_barrier()
            first = False

            @pl.when(s < 15)
            def _():
                pltpu.sync_copy(
                    z_hbm.at[pl.ds(chunk * N + h * HALF + s * 312, 312)],
                    acc.at[pl.ds(s * 312, 312)])

            @pl.when(s == 15)
            def _():
                pltpu.sync_copy(
                    z_hbm.at[pl.ds(chunk * N + h * HALF + 4680, 320)],
                    acc.at[pl.ds(4680, 320)])
                pltpu.sync_copy(zrows_hbm, acc.at[pl.ds(HALF, NH - HALF)])

            plsc.subcore_barrier()

            srcb = (src_v0, src_v1)
            dstb = (dst_v0, dst_v1)
            rowsb = (rows_v0, rows_v1)
            ebase = s * ept
            ih = (
                pltpu.async_copy(src4_hbm.at[pl.ds(chunk * EP + ebase, W)],
                                 srcb[0], isem),
                pltpu.async_copy(dh_hbm.at[pl.ds(ebase, W)], dstb[0], isem),
            )
            scat = [None, None]
            for w in range(nwin):
                b = w & 1
                ih[0].wait()
                ih[1].wait()
                if scat[b] is not None:
                    scat[b].wait()
                    scat[b] = None
                pltpu.async_copy(z_hbm.at[srcb[b]], rowsb[b], gsem).wait()
                scat[b] = pltpu.async_copy(rowsb[b], acc.at[dstb[b]], ssem,
                                           add=True)
                if w + 1 < nwin:
                    if scat[1 - b] is not None:
                        scat[1 - b].wait()
                        scat[1 - b] = None
                    eoff = ebase + (w + 1) * W
                    ih = (
                        pltpu.async_copy(
                            src4_hbm.at[pl.ds(chunk * EP + eoff, W)],
                            srcb[1 - b], isem),
                        pltpu.async_copy(dh_hbm.at[pl.ds(eoff, W)],
                                         dstb[1 - b], isem),
                    )
            for sh in scat:
                if sh is not None:
                    sh.wait()

            plsc.subcore_barrier()

            @pl.when(s < 15)
            def _():
                pltpu.sync_copy(
                    acc.at[pl.ds(s * 312, 312)],
                    y_hbm.at[pl.ds(chunk * N + h * HALF + s * 312, 312)])

            @pl.when(s == 15)
            def _():
                pltpu.sync_copy(
                    acc.at[pl.ds(4680, 320)],
                    y_hbm.at[pl.ds(chunk * N + h * HALF + 4680, 320)])


def _tc1_kernel(x_ref, w_ref, b_ref, deg_ref, z_ref, dinv_ref):
    deg = deg_ref[...] + 1.0
    dinv = lax.rsqrt(jnp.maximum(deg, 1e-6))
    acc = jnp.dot(x_ref[...], w_ref[0], preferred_element_type=jnp.float32)
    z_ref[0] = dinv * (acc + b_ref[0])
    dinv_ref[...] = dinv


def _tc1(x, w, b, deg):
    return pl.pallas_call(
        _tc1_kernel,
        grid=(N // BN, NCHUNK),
        in_specs=[
            pl.BlockSpec((BN, D_IN), lambda i, cc: (i, 0)),
            pl.BlockSpec((1, D_IN, CW), lambda i, cc: (cc, 0, 0)),
            pl.BlockSpec((1, 1, CW), lambda i, cc: (cc, 0, 0)),
            pl.BlockSpec((BN, 1), lambda i, cc: (i, 0)),
        ],
        out_specs=[
            pl.BlockSpec((1, BN, CW), lambda i, cc: (cc, i, 0)),
            pl.BlockSpec((BN, 1), lambda i, cc: (i, 0)),
        ],
        out_shape=[
            jax.ShapeDtypeStruct((NCHUNK, N, CW), jnp.float32),
            jax.ShapeDtypeStruct((N, 1), jnp.float32),
        ],
    )(x, w, b, deg)


def _tc2_kernel(y_ref, w_ref, b_ref, dinv_ref, z_ref):
    k = pl.program_id(2)
    dinv = dinv_ref[...]
    h = jnp.maximum(dinv * y_ref[0], 0.0)
    part = jnp.dot(h, w_ref[0, 0], preferred_element_type=jnp.float32)

    @pl.when(k == 0)
    def _():
        z_ref[0] = part + b_ref[0]

    @pl.when(k > 0)
    def _():
        z_ref[0] += part

    @pl.when(k == NCHUNK - 1)
    def _():
        z_ref[0] *= dinv


def _tc2(y, w, b, dinv):
    return pl.pallas_call(
        _tc2_kernel,
        grid=(N // BN, NCHUNK, NCHUNK),
        in_specs=[
            pl.BlockSpec((1, BN, CW), lambda i, co, k: (k, i, 0)),
            pl.BlockSpec((1, 1, CW, CW), lambda i, co, k: (k, co, 0, 0)),
            pl.BlockSpec((1, 1, CW), lambda i, co, k: (co, 0, 0)),
            pl.BlockSpec((BN, 1), lambda i, co, k: (i, 0)),
        ],
        out_specs=pl.BlockSpec((1, BN, CW), lambda i, co, k: (co, i, 0)),
        out_shape=jax.ShapeDtypeStruct((NCHUNK, N, CW), jnp.float32),
    )(y, w, b, dinv)


def _tc3_kernel(y_ref, w_ref, b_ref, dinv_ref, o_ref):
    k = pl.program_id(1)
    h = jnp.maximum(dinv_ref[...] * y_ref[0], 0.0)
    part = jnp.dot(h, w_ref[0], preferred_element_type=jnp.float32)

    @pl.when(k == 0)
    def _():
        o_ref[...] = part + b_ref[...]

    @pl.when(k > 0)
    def _():
        o_ref[...] += part


def _tc3(y, w, b, dinv):
    return pl.pallas_call(
        _tc3_kernel,
        grid=(N // BN, NCHUNK),
        in_specs=[
            pl.BlockSpec((1, BN, CW), lambda i, k: (k, i, 0)),
            pl.BlockSpec((1, CW, N_CLS), lambda i, k: (k, 0, 0)),
            pl.BlockSpec((1, N_CLS), lambda i, k: (0, 0)),
            pl.BlockSpec((BN, 1), lambda i, k: (i, 0)),
        ],
        out_specs=pl.BlockSpec((BN, N_CLS), lambda i, k: (i, 0)),
        out_shape=jax.ShapeDtypeStruct((N, N_CLS), jnp.float32),
    )(y, w, b, dinv)


def kernel(x, edge_index, W1, b1, W2, b2, Wd, bd):
    src = edge_index[0].astype(jnp.int32)
    dst = edge_index[1].astype(jnp.int32)

    padc = EP - E
    pidx = jnp.arange(padc, dtype=jnp.int32)
    eidx = jnp.arange(EP, dtype=jnp.int32)
    src_p = jnp.concatenate([src, (pidx * 97) % N])
    dst_p = jnp.concatenate([dst, jnp.full((padc,), -1, jnp.int32)])
    dummy = HALF + eidx % (NH - HALF)
    dh0 = jnp.where((dst_p >= 0) & (dst_p < HALF), dst_p, dummy)
    dh1 = jnp.where(dst_p >= HALF, dst_p - HALF, dummy)
    offs = (jnp.arange(NCHUNK, dtype=jnp.int32) * N)[:, None]
    src4 = (src_p[None, :] + offs).reshape(-1)
    zrows = jnp.zeros((NH - HALF, CW), jnp.float32)

    W1r = W1.reshape(D_IN, NCHUNK, CW).transpose(1, 0, 2)
    W2r = W2.reshape(NCHUNK, CW, NCHUNK, CW).transpose(0, 2, 1, 3)

    deg = _deg_kernel(dst)[:N].reshape(N, 1)
    z1, dinv = _tc1(x, W1r, b1.reshape(NCHUNK, 1, CW), deg)
    y1 = _prop_kernel(z1.reshape(NCHUNK * N, CW), src4, dh0, dh1, zrows)
    z2 = _tc2(y1.reshape(NCHUNK, N, CW), W2r, b2.reshape(NCHUNK, 1, CW), dinv)
    y2 = _prop_kernel(z2.reshape(NCHUNK * N, CW), src4, dh0, dh1, zrows)
    logits = _tc3(y2.reshape(NCHUNK, N, CW), Wd.reshape(NCHUNK, CW, N_CLS),
                  bd.reshape(1, N_CLS), dinv)
    return logits
